# baseline (speedup 1.0000x reference)
import functools
import jax
import jax.numpy as jnp
from jax import lax
from jax.experimental import pallas as pl
from jax.experimental.pallas import tpu as pltpu
from jax.experimental.pallas import tpu_sc as plsc

B, C, H, W = 16, 384, 64, 64
D = H * W
R = B * C
NC, NS = 2, 16
NW = NC * NS
RPW = R // NW
CPW = C // NC
CHUNK = 8
NCHUNK = RPW // CHUNK
L = 16


def _body(x_hbm, perm_hbm, out_hbm, perm_v, idx_v, buf, gsem):
    wid = lax.axis_index("s") * NC + lax.axis_index("c")
    b = wid // 2
    c0 = (wid % 2) * CPW
    base = wid * RPW

    pltpu.sync_copy(perm_hbm, perm_v)
    for j in range(RPW // L):
        idx_v[pl.ds(L * j, L)] = perm_v[pl.ds(c0 + L * j, L)] + b * C

    def chunk_body(k, carry):
        pltpu.async_copy(
            x_hbm.at[idx_v.at[pl.ds(k * CHUNK, CHUNK)]], buf.at[0], gsem
        ).wait()
        pltpu.sync_copy(buf.at[0], out_hbm.at[pl.ds(base + k * CHUNK, CHUNK)])
        return carry

    lax.fori_loop(0, NCHUNK, chunk_body, 0)


@jax.jit
def _shuffle(x2d, perm32):
    mesh = plsc.VectorSubcoreMesh(
        core_axis_name="c", subcore_axis_name="s", num_cores=NC, num_subcores=NS
    )
    f = pl.kernel(
        _body,
        out_type=jax.ShapeDtypeStruct((R, D), jnp.float32),
        mesh=mesh,
        scratch_types=[
            pltpu.VMEM((C,), jnp.int32),
            pltpu.VMEM((RPW,), jnp.int32),
            pltpu.VMEM((1, CHUNK, D), jnp.float32),
            pltpu.SemaphoreType.DMA,
        ],
    )
    return f(x2d, perm32)


def kernel(inputs, permutation):
    x2d = inputs.reshape(R, D)
    perm32 = permutation.astype(jnp.int32)
    return _shuffle(x2d, perm32).reshape(B, C, H, W)


# speedup vs baseline: 1.0270x; 1.0270x over previous
import jax
import jax.numpy as jnp
from jax import lax
from jax.experimental import pallas as pl
from jax.experimental.pallas import tpu as pltpu
from jax.experimental.pallas import tpu_sc as plsc

B, C, H, W = 16, 384, 64, 64
D = H * W
R = B * C
NC, NS = 2, 16
NW = NC * NS
RPW = R // NW
CPW = C // NC
CHUNK = 8
NCHUNK = RPW // CHUNK
NPAIR = NCHUNK // 2
L = 16


def _body(x_hbm, perm_hbm, out_hbm, perm_v, idx_v, buf, gs0, gs1, ss0, ss1):
    wid = lax.axis_index("s") * NC + lax.axis_index("c")
    b = wid // 2
    c0 = (wid % 2) * CPW
    base = wid * RPW

    pltpu.sync_copy(perm_hbm, perm_v)
    for j in range(RPW // L):
        idx_v[pl.ds(L * j, L)] = perm_v[pl.ds(c0 + L * j, L)] + b * C

    def gather(k, slot, sem):
        return pltpu.make_async_copy(
            x_hbm.at[idx_v.at[pl.ds(k * CHUNK, CHUNK)]], buf.at[slot], sem
        )

    def scatter(k, slot, sem):
        return pltpu.make_async_copy(
            buf.at[slot], out_hbm.at[pl.ds(base + k * CHUNK, CHUNK)], sem
        )

    gather(0, 0, gs0).start()

    def pair(i, carry):
        k0 = 2 * i
        k1 = k0 + 1
        gather(k0, 0, gs0).wait()

        @pl.when(i > 0)
        def _():
            scatter(k0 - 1, 1, ss1).wait()

        scatter(k0, 0, ss0).start()
        gather(k1, 1, gs1).start()
        gather(k1, 1, gs1).wait()
        scatter(k0, 0, ss0).wait()
        scatter(k1, 1, ss1).start()

        @pl.when(i < NPAIR - 1)
        def _():
            gather(k0 + 2, 0, gs0).start()

        return carry

    lax.fori_loop(0, NPAIR, pair, 0)
    scatter(NCHUNK - 1, 1, ss1).wait()


@jax.jit
def _shuffle(x2d, perm32):
    mesh = plsc.VectorSubcoreMesh(
        core_axis_name="c", subcore_axis_name="s", num_cores=NC, num_subcores=NS
    )
    f = pl.kernel(
        _body,
        out_type=jax.ShapeDtypeStruct((R, D), jnp.float32),
        mesh=mesh,
        scratch_types=[
            pltpu.VMEM((C,), jnp.int32),
            pltpu.VMEM((RPW,), jnp.int32),
            pltpu.VMEM((2, CHUNK, D), jnp.float32),
            pltpu.SemaphoreType.DMA,
            pltpu.SemaphoreType.DMA,
            pltpu.SemaphoreType.DMA,
            pltpu.SemaphoreType.DMA,
        ],
    )
    return f(x2d, perm32)


def kernel(inputs, permutation):
    x2d = inputs.reshape(R, D)
    perm32 = permutation.astype(jnp.int32)
    return _shuffle(x2d, perm32).reshape(B, C, H, W)


# speedup vs baseline: 1.0292x; 1.0022x over previous
import jax
import jax.numpy as jnp
from jax import lax
from jax.experimental import pallas as pl
from jax.experimental.pallas import tpu as pltpu
from jax.experimental.pallas import tpu_sc as plsc

B, C, H, W = 16, 384, 64, 64
D = H * W
R = B * C
NC, NS = 2, 16
NW = NC * NS
RPW = R // NW
CPW = C // NC
CHUNK = 8
NCHUNK = RPW // CHUNK
NTRI = NCHUNK // 3
L = 16


def _body(x_hbm, perm_hbm, out_hbm, perm_v, idx_v, buf, g0, g1, g2, s0, s1, s2):
    wid = lax.axis_index("s") * NC + lax.axis_index("c")
    b = wid // 2
    c0 = (wid % 2) * CPW
    base = wid * RPW

    pltpu.sync_copy(perm_hbm, perm_v)
    for j in range(RPW // L):
        idx_v[pl.ds(L * j, L)] = perm_v[pl.ds(c0 + L * j, L)] + b * C

    gsems = (g0, g1, g2)
    ssems = (s0, s1, s2)

    def gather(k, slot):
        return pltpu.make_async_copy(
            x_hbm.at[idx_v.at[pl.ds(k * CHUNK, CHUNK)]], buf.at[slot], gsems[slot]
        )

    def scatter(k, slot):
        return pltpu.make_async_copy(
            buf.at[slot], out_hbm.at[pl.ds(base + k * CHUNK, CHUNK)], ssems[slot]
        )

    gather(0, 0).start()
    gather(1, 1).start()

    def tri(i, carry):
        for j in range(3):
            k = 3 * i + j
            sg = j
            sn = (j + 2) % 3

            gather(k, sg).wait()
            scatter(k, sg).start()

            if j == 0:

                @pl.when(i > 0)
                def _():
                    scatter(k - 1, sn).wait()

                gather(k + 2, sn).start()
            else:
                scatter(k - 1, sn).wait()

                @pl.when(i < NTRI - 1)
                def _():
                    gather(k + 2, sn).start()

        return carry

    lax.fori_loop(0, NTRI, tri, 0)
    scatter(NCHUNK - 1, 2).wait()


@jax.jit
def _shuffle(x2d, perm32):
    mesh = plsc.VectorSubcoreMesh(
        core_axis_name="c", subcore_axis_name="s", num_cores=NC, num_subcores=NS
    )
    f = pl.kernel(
        _body,
        out_type=jax.ShapeDtypeStruct((R, D), jnp.float32),
        mesh=mesh,
        scratch_types=[
            pltpu.VMEM((C,), jnp.int32),
            pltpu.VMEM((RPW,), jnp.int32),
            pltpu.VMEM((3, CHUNK, D), jnp.float32),
            pltpu.SemaphoreType.DMA,
            pltpu.SemaphoreType.DMA,
            pltpu.SemaphoreType.DMA,
            pltpu.SemaphoreType.DMA,
            pltpu.SemaphoreType.DMA,
            pltpu.SemaphoreType.DMA,
        ],
    )
    return f(x2d, perm32)


def kernel(inputs, permutation):
    x2d = inputs.reshape(R, D)
    perm32 = permutation.astype(jnp.int32)
    return _shuffle(x2d, perm32).reshape(B, C, H, W)
